# baseline (speedup 1.0000x reference)
import functools
import jax
import jax.numpy as jnp
from jax import lax
from jax.experimental import pallas as pl
from jax.experimental.pallas import tpu as pltpu
from jax.experimental.pallas import tpu_sc as plsc

N_M = 4096
N_D = 4096
N = N_M + N_D
HIDDEN = 256
HEADS = 4
C1 = 128
C2 = 64
E_M = 131072
E_D = 131072
E = 262144


_SC_MESH = plsc.VectorSubcoreMesh(core_axis_name="c", subcore_axis_name="s")
_SC_PARAMS = pltpu.CompilerParams(needs_layout_passes=False)


def _ew_deg_body(mm_ref, md_ref, srcm_ref, dstm_ref, srcd_ref, dstd_ref,
                 ewm_ref, ewd_ref, degp_ref,
                 src_b, dst_b, idx_b, ew_b, deg_l, sem):
    cid = lax.axis_index("c")
    sid = lax.axis_index("s")
    z16 = jnp.zeros((16,), jnp.float32)

    def zbody(i, _):
        deg_l[pl.ds(i * 16, 16)] = z16
        return 0
    lax.fori_loop(0, N_M // 16, zbody, 0)

    def _process(m_ref, src_ref, dst_ref, ew_ref):
        base_r = sid * 64
        pltpu.sync_copy(src_ref.at[pl.ds(base_r, 64)], src_b)
        pltpu.sync_copy(dst_ref.at[pl.ds(base_r, 64)], dst_b)

        def ibody(j, _):
            for k in range(8):
                s16 = src_b[j, pl.ds(k * 16, 16)]
                d16 = dst_b[j, pl.ds(k * 16, 16)]
                idx_b[j, pl.ds(k * 16, 16)] = s16 * N_M + d16
            return 0
        lax.fori_loop(0, 64, ibody, 0)

        def gbody(j, _):
            pltpu.async_copy(m_ref.at[idx_b.at[j]], ew_b.at[j], sem).wait()
            return 0
        lax.fori_loop(0, 64, gbody, 0)

        pltpu.sync_copy(ew_b, ew_ref.at[pl.ds(base_r, 64)])

        def dbody(j, _):
            for k in range(8):
                d16 = dst_b[j, pl.ds(k * 16, 16)]
                e16 = ew_b[j, pl.ds(k * 16, 16)]
                plsc.addupdate_scatter(deg_l, [d16], e16)
            return 0
        lax.fori_loop(0, 64, dbody, 0)

    plsc.subcore_barrier()

    @pl.when(cid == 0)
    def _():
        _process(mm_ref, srcm_ref, dstm_ref, ewm_ref)

    @pl.when(cid == 1)
    def _():
        _process(md_ref, srcd_ref, dstd_ref, ewd_ref)

    pltpu.sync_copy(deg_l, degp_ref.at[cid, sid])


def _sc_ew_deg(mm_flat, md_flat, srcm, dstm, srcd, dstd):
    f = pl.kernel(
        _ew_deg_body,
        out_type=(
            jax.ShapeDtypeStruct((E_M // 128, 128), jnp.float32),
            jax.ShapeDtypeStruct((E_D // 128, 128), jnp.float32),
            jax.ShapeDtypeStruct((2, 16, N_M), jnp.float32),
        ),
        mesh=_SC_MESH,
        scratch_types=[
            pltpu.VMEM((64, 128), jnp.int32),
            pltpu.VMEM((64, 128), jnp.int32),
            pltpu.VMEM((64, 128), jnp.int32),
            pltpu.VMEM((64, 128), jnp.float32),
            pltpu.VMEM((N_M,), jnp.float32),
            pltpu.SemaphoreType.DMA,
        ],
        compiler_params=_SC_PARAMS,
    )
    return f(mm_flat, md_flat, srcm, dstm, srcd, dstd)


_CAP = 10240


def _gcn_agg_body(xwm_ref, xwd_ref, srcm_ref, dstm_ref, ewm_ref,
                  srcd_ref, dstd_ref, ewd_ref, dinvm_ref, dinvd_ref,
                  part_ref,
                  src_b, dst_b, ew_b, sloc, dloc, nloc, dinv_l, rows, rows2,
                  acc, sem, sem2):
    cid = lax.axis_index("c")
    sid = lax.axis_index("s")
    z16 = jnp.zeros((16,), jnp.float32)
    zi16 = jnp.zeros((16,), jnp.int32)

    def zb(i, _):
        sloc[pl.ds(i * 16, 16)] = zi16
        dloc[pl.ds(i * 16, 16)] = zi16
        nloc[pl.ds(i * 16, 16)] = z16
        return 0
    lax.fori_loop(0, _CAP // 16, zb, 0)

    def za(r, _):
        for k in range(HIDDEN // 16):
            acc[r, pl.ds(k * 16, 16)] = z16
        return 0
    lax.fori_loop(0, 256, za, 0)

    def _process(xw_ref, src_ref, dst_ref, ew_ref, dinv_ref):
        pltpu.sync_copy(dinv_ref, dinv_l)
        lo = sid * 256

        def obody(o, off):
            d1 = pltpu.async_copy(src_ref.at[pl.ds(o * 32, 32)], src_b, sem)
            d2 = pltpu.async_copy(dst_ref.at[pl.ds(o * 32, 32)], dst_b, sem2)
            d3 = pltpu.async_copy(ew_ref.at[pl.ds(o * 32, 32)], ew_b, sem)
            d1.wait()
            d2.wait()
            d3.wait()

            def cbody(g, off):
                j = g // 8
                k = g % 8
                s16 = src_b[j, pl.ds(k * 16, 16)]
                d16 = dst_b[j, pl.ds(k * 16, 16)]
                e16 = ew_b[j, pl.ds(k * 16, 16)]
                msk = (d16 >> 8) == sid
                plsc.store_compressed(sloc.at[pl.ds(off, 16)], s16, mask=msk)
                plsc.store_compressed(dloc.at[pl.ds(off, 16)], d16 & 255, mask=msk)
                plsc.store_compressed(nloc.at[pl.ds(off, 16)], e16, mask=msk)
                cnt = plsc.all_reduce_population_count(msk)[0]
                return off + cnt
            return lax.fori_loop(0, 256, cbody, off)
        ne = lax.fori_loop(0, 32, obody, 0)

        nb = (ne + 15) >> 4

        def nbody(g, _):
            s16 = sloc[pl.ds(g * 16, 16)]
            d16 = dloc[pl.ds(g * 16, 16)] + lo
            e16 = nloc[pl.ds(g * 16, 16)]
            di_s = plsc.load_gather(dinv_l, [s16])
            di_d = plsc.load_gather(dinv_l, [d16])
            nloc[pl.ds(g * 16, 16)] = di_s * e16 * di_d
            return 0
        lax.fori_loop(0, nb, nbody, 0)

        nbat = (ne + 31) >> 5

        def bproc(buf, b):
            def gbody(g, _):
                n16 = nloc[pl.ds(b * 32 + g * 16, 16)]
                d16 = dloc[pl.ds(b * 32 + g * 16, 16)]
                for i in range(16):
                    r = g * 16 + i
                    s = n16[i]
                    dl = d16[i]
                    for k in range(HIDDEN // 16):
                        acc[dl, pl.ds(k * 16, 16)] = (
                            acc[dl, pl.ds(k * 16, 16)]
                            + buf[r, pl.ds(k * 16, 16)] * s)
                return 0
            lax.fori_loop(0, 2, gbody, 0)

        @pl.when(nbat > 0)
        def _():
            pltpu.async_copy(xw_ref.at[sloc.at[pl.ds(0, 32)]], rows, sem)

        def bpair(p, _):
            b0 = 2 * p
            b1 = b0 + 1

            @pl.when(b1 < nbat)
            def _():
                pltpu.async_copy(
                    xw_ref.at[sloc.at[pl.ds(b1 * 32, 32)]], rows2, sem2)
            pltpu.make_async_copy(
                xw_ref.at[sloc.at[pl.ds(b0 * 32, 32)]], rows, sem).wait()
            bproc(rows, b0)

            @pl.when(b1 < nbat)
            def _():
                @pl.when(b1 + 1 < nbat)
                def _():
                    pltpu.async_copy(
                        xw_ref.at[sloc.at[pl.ds((b1 + 1) * 32, 32)]],
                        rows, sem)
                pltpu.make_async_copy(
                    xw_ref.at[sloc.at[pl.ds(b1 * 32, 32)]], rows2, sem2).wait()
                bproc(rows2, b1)
            return 0
        lax.fori_loop(0, (nbat + 1) >> 1, bpair, 0)

    @pl.when(cid == 0)
    def _():
        _process(xwm_ref, srcm_ref, dstm_ref, ewm_ref, dinvm_ref)

    @pl.when(cid == 1)
    def _():
        _process(xwd_ref, srcd_ref, dstd_ref, ewd_ref, dinvd_ref)

    pltpu.sync_copy(acc, part_ref.at[cid, pl.ds(sid * 256, 256)])


def _sc_gcn_agg(xw_m, xw_d, srcm, dstm, ewm, srcd, dstd, ewd, dinv_m, dinv_d):
    f = pl.kernel(
        _gcn_agg_body,
        out_type=jax.ShapeDtypeStruct((2, N_M, HIDDEN), jnp.float32),
        mesh=_SC_MESH,
        scratch_types=[
            pltpu.VMEM((32, 128), jnp.int32),
            pltpu.VMEM((32, 128), jnp.int32),
            pltpu.VMEM((32, 128), jnp.float32),
            pltpu.VMEM((_CAP,), jnp.int32),
            pltpu.VMEM((_CAP,), jnp.int32),
            pltpu.VMEM((_CAP,), jnp.float32),
            pltpu.VMEM((N_M,), jnp.float32),
            pltpu.VMEM((32, HIDDEN), jnp.float32),
            pltpu.VMEM((32, HIDDEN), jnp.float32),
            pltpu.VMEM((256, HIDDEN), jnp.float32),
            pltpu.SemaphoreType.DMA,
            pltpu.SemaphoreType.DMA,
        ],
        compiler_params=_SC_PARAMS,
    )
    return f(xw_m, xw_d, srcm, dstm, ewm, srcd, dstd, ewd, dinv_m, dinv_d)


_GCAP = 9216


def _make_gat_body(C):
    HC = HEADS * C

    def body(xl_ref, xr_ref, att_ref, src_ref, dst_ref, out_ref,
             src_b, dst_b, sloc, dloc, exl, den, att_l, xlr, xrr, acc,
             sem, sem2):
        cid = lax.axis_index("c")
        sid = lax.axis_index("s")
        wid = cid * 16 + sid
        lo = wid * 256
        z16 = jnp.zeros((16,), jnp.float32)
        zi16 = jnp.zeros((16,), jnp.int32)
        ninf = jnp.full((16,), -1e30, jnp.float32)
        lane0 = lax.iota(jnp.int32, 16) == 0

        pltpu.sync_copy(att_ref, att_l)

        def zb(i, _):
            sloc[pl.ds(i * 16, 16)] = zi16
            dloc[pl.ds(i * 16, 16)] = zi16
            return 0
        lax.fori_loop(0, _GCAP // 16, zb, 0)

        def ze(i, _):
            exl[pl.ds(i * 16, 16)] = ninf
            return 0
        lax.fori_loop(0, 4 * _GCAP // 16, ze, 0)

        def zd(i, _):
            den[pl.ds(i * 16, 16)] = z16
            return 0
        lax.fori_loop(0, 64, zd, 0)

        def za(r, _):
            for k in range(C // 16):
                acc[r, pl.ds(k * 16, 16)] = z16
            return 0
        lax.fori_loop(0, 256, za, 0)

        def obody(o, off):
            d1 = pltpu.async_copy(src_ref.at[pl.ds(o * 64, 64)], src_b, sem)
            d2 = pltpu.async_copy(dst_ref.at[pl.ds(o * 64, 64)], dst_b, sem2)
            d1.wait()
            d2.wait()

            def cbody(g, off):
                j = g // 2
                k = g % 2
                s16 = src_b[j, pl.ds(k * 16, 16)]
                d16 = dst_b[j, pl.ds(k * 16, 16)]
                msk = (d16 >> 8) == wid
                plsc.store_compressed(sloc.at[pl.ds(off, 16)], s16, mask=msk)
                plsc.store_compressed(dloc.at[pl.ds(off, 16)], d16, mask=msk)
                cnt = plsc.all_reduce_population_count(msk)[0]
                return off + cnt
            return lax.fori_loop(0, 128, cbody, off)
        ne = lax.fori_loop(0, 132, obody, 0)

        nbat = (ne + 15) >> 4

        def s1iss(b, bl, br, sm):
            pltpu.async_copy(xl_ref.at[sloc.at[pl.ds(b * 16, 16)]], bl, sm)
            pltpu.async_copy(xr_ref.at[dloc.at[pl.ds(b * 16, 16)]], br, sm)

        def s1drain(b, bl, br, sm):
            pltpu.make_async_copy(
                xl_ref.at[sloc.at[pl.ds(b * 16, 16)]], bl, sm).wait()
            pltpu.make_async_copy(
                xr_ref.at[dloc.at[pl.ds(b * 16, 16)]], br, sm).wait()

        def s1proc(bl, br, b):
            def ebody(r, _):
                e = b * 16 + r
                for h in range(HEADS):
                    t = z16
                    for k in range(C // 16):
                        o = h * C + k * 16
                        z = bl[r, pl.ds(o, 16)] + br[r, pl.ds(o, 16)]
                        l = jnp.maximum(z, z * 0.2)
                        t = t + l * att_l[pl.ds(o, 16)]
                    a = jnp.sum(t)
                    plsc.store_scatter(
                        exl, [jnp.broadcast_to(h * _GCAP + e, (16,))],
                        jnp.broadcast_to(a, (16,)), mask=lane0)
                return 0
            lax.fori_loop(0, 16, ebody, 0)

        def s1body(b, _):
            d1 = pltpu.async_copy(
                xl_ref.at[sloc.at[pl.ds(b * 16, 16)]], xlr, sem)
            d2 = pltpu.async_copy(
                xr_ref.at[dloc.at[pl.ds(b * 16, 16)]], xrr, sem2)
            d1.wait()
            d2.wait()
            s1proc(xlr, xrr, b)
            return 0
        lax.fori_loop(0, nbat, s1body, 0)

        ng = (ne + 15) >> 4

        def exbody(g, _):
            dl16 = dloc[pl.ds(g * 16, 16)] & 255
            for h in range(HEADS):
                ex16 = jnp.exp(exl[pl.ds(h * _GCAP + g * 16, 16)])
                exl[pl.ds(h * _GCAP + g * 16, 16)] = ex16
                plsc.addupdate_scatter(den, [h * 256 + dl16], ex16)
            return 0
        lax.fori_loop(0, ng, exbody, 0)

        def rdbody(g, _):
            d16 = den[pl.ds(g * 16, 16)]
            den[pl.ds(g * 16, 16)] = 1.0 / (d16 + 1e-16)
            return 0
        lax.fori_loop(0, 64, rdbody, 0)

        def s2proc(buf, b):
            base = b * 16
            dl16 = dloc[pl.ds(base, 16)] & 255
            av = []
            for h in range(HEADS):
                rd = plsc.load_gather(den, [h * 256 + dl16])
                av.append(exl[pl.ds(h * _GCAP + base, 16)] * rd)
            for i in range(16):
                dl = dl16[i]
                s0 = av[0][i]
                s1 = av[1][i]
                s2 = av[2][i]
                s3 = av[3][i]
                for k in range(C // 16):
                    acc[dl, pl.ds(k * 16, 16)] = (
                        acc[dl, pl.ds(k * 16, 16)]
                        + buf[i, pl.ds(0 * C + k * 16, 16)] * s0
                        + buf[i, pl.ds(1 * C + k * 16, 16)] * s1
                        + buf[i, pl.ds(2 * C + k * 16, 16)] * s2
                        + buf[i, pl.ds(3 * C + k * 16, 16)] * s3)

        @pl.when(nbat > 0)
        def _():
            pltpu.async_copy(xl_ref.at[sloc.at[pl.ds(0, 16)]], xlr, sem)

        def s2pair(p, _):
            b0 = 2 * p
            b1 = b0 + 1

            @pl.when(b1 < nbat)
            def _():
                pltpu.async_copy(
                    xl_ref.at[sloc.at[pl.ds(b1 * 16, 16)]], xrr, sem2)
            pltpu.make_async_copy(
                xl_ref.at[sloc.at[pl.ds(b0 * 16, 16)]], xlr, sem).wait()
            s2proc(xlr, b0)

            @pl.when(b1 < nbat)
            def _():
                @pl.when(b1 + 1 < nbat)
                def _():
                    pltpu.async_copy(
                        xl_ref.at[sloc.at[pl.ds((b1 + 1) * 16, 16)]],
                        xlr, sem)
                pltpu.make_async_copy(
                    xl_ref.at[sloc.at[pl.ds(b1 * 16, 16)]], xrr, sem2).wait()
                s2proc(xrr, b1)
            return 0
        lax.fori_loop(0, (nbat + 1) >> 1, s2pair, 0)

        pltpu.sync_copy(acc, out_ref.at[pl.ds(lo, 256)])

    return body


def _sc_gat(xl, xr, att, src2, dst2, C):
    f = pl.kernel(
        _make_gat_body(C),
        out_type=jax.ShapeDtypeStruct((N, C), jnp.float32),
        mesh=_SC_MESH,
        scratch_types=[
            pltpu.VMEM((64, 32), jnp.int32),
            pltpu.VMEM((64, 32), jnp.int32),
            pltpu.VMEM((_GCAP,), jnp.int32),
            pltpu.VMEM((_GCAP,), jnp.int32),
            pltpu.VMEM((4 * _GCAP,), jnp.float32),
            pltpu.VMEM((1024,), jnp.float32),
            pltpu.VMEM((HEADS * C,), jnp.float32),
            pltpu.VMEM((16, HEADS * C), jnp.float32),
            pltpu.VMEM((16, HEADS * C), jnp.float32),
            pltpu.VMEM((256, C), jnp.float32),
            pltpu.SemaphoreType.DMA,
            pltpu.SemaphoreType.DMA,
        ],
        compiler_params=_SC_PARAMS,
    )
    return f(xl, xr, att, src2, dst2)


def _gat2_body(xl_ref, xr_ref, att_ref, src_ref, dst_ref, out_ref,
               src_b, dst_b, sloc, dloc, den, att_l, xlr, xrr, xlr2, xrr2,
               acch, outb, sem, sem2):
    C = C2
    cid = lax.axis_index("c")
    sid = lax.axis_index("s")
    wid = cid * 16 + sid
    lo = wid * 256
    z16 = jnp.zeros((16,), jnp.float32)
    zi16 = jnp.zeros((16,), jnp.int32)
    iot = lax.iota(jnp.int32, 16)
    m4 = iot < 4

    pltpu.sync_copy(att_ref, att_l)

    def zb(i, _):
        sloc[pl.ds(i * 16, 16)] = zi16
        dloc[pl.ds(i * 16, 16)] = zi16
        return 0
    lax.fori_loop(0, _GCAP // 16, zb, 0)

    def zd(i, _):
        den[pl.ds(i * 16, 16)] = z16
        return 0
    lax.fori_loop(0, 64, zd, 0)

    def za(r, _):
        for k in range(16):
            acch[r, pl.ds(k * 16, 16)] = z16
        return 0
    lax.fori_loop(0, 256, za, 0)

    def obody(o, off):
        d1 = pltpu.async_copy(src_ref.at[pl.ds(o * 64, 64)], src_b, sem)
        d2 = pltpu.async_copy(dst_ref.at[pl.ds(o * 64, 64)], dst_b, sem2)
        d1.wait()
        d2.wait()

        def cbody(g, off):
            j = g // 2
            k = g % 2
            s16 = src_b[j, pl.ds(k * 16, 16)]
            d16 = dst_b[j, pl.ds(k * 16, 16)]
            msk = (d16 >> 8) == wid
            plsc.store_compressed(sloc.at[pl.ds(off, 16)], s16, mask=msk)
            plsc.store_compressed(dloc.at[pl.ds(off, 16)], d16, mask=msk)
            cnt = plsc.all_reduce_population_count(msk)[0]
            return off + cnt
        return lax.fori_loop(0, 128, cbody, off)
    ne = lax.fori_loop(0, 132, obody, 0)

    nbat = (ne + 7) >> 3

    def giss(b, bl, br, sm):
        pltpu.async_copy(xl_ref.at[sloc.at[pl.ds(b * 8, 8)]], bl, sm)
        pltpu.async_copy(xr_ref.at[dloc.at[pl.ds(b * 8, 8)]], br, sm)

    def gdrain(b, bl, br, sm):
        pltpu.make_async_copy(
            xl_ref.at[sloc.at[pl.ds(b * 8, 8)]], bl, sm).wait()
        pltpu.make_async_copy(
            xr_ref.at[dloc.at[pl.ds(b * 8, 8)]], br, sm).wait()

    def gproc(bl, br, b):
        base = b * 8
        dl16 = dloc[pl.ds(base, 16)] & 255
        for i in range(8):
            e = base + i
            dl = dl16[i]

            @pl.when(e < ne)
            def _(dl=dl, i=i):
                av = z16
                for h in range(HEADS):
                    t = z16
                    for k in range(C // 16):
                        o = h * C + k * 16
                        z = bl[i, pl.ds(o, 16)] + br[i, pl.ds(o, 16)]
                        lk = jnp.maximum(z, z * 0.2)
                        t = t + lk * att_l[pl.ds(o, 16)]
                    av = jnp.where(iot == h, jnp.sum(t), av)
                exv = jnp.exp(av)
                plsc.addupdate_scatter(den, [iot * 256 + dl], exv, mask=m4)
                for h in range(HEADS):
                    eh = exv[h]
                    for k in range(C // 16):
                        o = h * C + k * 16
                        acch[dl, pl.ds(o, 16)] = (
                            acch[dl, pl.ds(o, 16)] + bl[i, pl.ds(o, 16)] * eh)

    @pl.when(nbat > 0)
    def _():
        giss(0, xlr, xrr, sem)

    def gpair(p, _):
        b0 = 2 * p
        b1 = b0 + 1

        @pl.when(b1 < nbat)
        def _():
            giss(b1, xlr2, xrr2, sem2)
        gdrain(b0, xlr, xrr, sem)
        gproc(xlr, xrr, b0)

        @pl.when(b1 < nbat)
        def _():
            @pl.when(b1 + 1 < nbat)
            def _():
                giss(b1 + 1, xlr, xrr, sem)
            gdrain(b1, xlr2, xrr2, sem2)
            gproc(xlr2, xrr2, b1)
        return 0
    lax.fori_loop(0, (nbat + 1) >> 1, gpair, 0)

    def rdbody(g, _):
        d16 = den[pl.ds(g * 16, 16)]
        den[pl.ds(g * 16, 16)] = 1.0 / (d16 + 1e-16)
        return 0
    lax.fori_loop(0, 64, rdbody, 0)

    def fin(dl, _):
        rd4 = plsc.load_gather(den, [iot * 256 + dl], mask=m4)
        r0 = rd4[0]
        r1 = rd4[1]
        r2 = rd4[2]
        r3 = rd4[3]
        for k in range(C // 16):
            outb[pl.ds(dl * C + k * 16, 16)] = (
                acch[dl, pl.ds(0 * C + k * 16, 16)] * r0
                + acch[dl, pl.ds(1 * C + k * 16, 16)] * r1
                + acch[dl, pl.ds(2 * C + k * 16, 16)] * r2
                + acch[dl, pl.ds(3 * C + k * 16, 16)] * r3)
        return 0
    lax.fori_loop(0, 256, fin, 0)

    pltpu.sync_copy(outb, out_ref.at[pl.ds(wid * 256 * C, 256 * C)])


def _sc_gat2(xl, xr, att, src2, dst2):
    f = pl.kernel(
        _gat2_body,
        out_type=jax.ShapeDtypeStruct((N * C2,), jnp.float32),
        mesh=_SC_MESH,
        scratch_types=[
            pltpu.VMEM((64, 32), jnp.int32),
            pltpu.VMEM((64, 32), jnp.int32),
            pltpu.VMEM((_GCAP,), jnp.int32),
            pltpu.VMEM((_GCAP,), jnp.int32),
            pltpu.VMEM((1024,), jnp.float32),
            pltpu.VMEM((HEADS * C2,), jnp.float32),
            pltpu.VMEM((8, HEADS * C2), jnp.float32),
            pltpu.VMEM((8, HEADS * C2), jnp.float32),
            pltpu.VMEM((8, HEADS * C2), jnp.float32),
            pltpu.VMEM((8, HEADS * C2), jnp.float32),
            pltpu.VMEM((256, HEADS * C2), jnp.float32),
            pltpu.VMEM((256 * C2,), jnp.float32),
            pltpu.SemaphoreType.DMA,
            pltpu.SemaphoreType.DMA,
        ],
        compiler_params=_SC_PARAMS,
    )
    return f(xl, xr, att, src2, dst2)


def _rsqrt_body(d_ref, o_ref):
    d = jnp.sum(d_ref[...], axis=0, keepdims=True) + 1.0
    o_ref[...] = jnp.where(d > 0, lax.rsqrt(d), 0.0)


def _tc_dinv(degp):
    return pl.pallas_call(
        _rsqrt_body,
        out_shape=jax.ShapeDtypeStruct((1, N_M), jnp.float32),
    )(degp)


def _mm_body(x_ref, w_ref, b_ref, o_ref):
    o_ref[...] = (
        jnp.dot(x_ref[...], w_ref[...], preferred_element_type=jnp.float32)
        + b_ref[...]
    )


def _tc_matmul_bias(x, w, b):
    m, k = x.shape
    n = w.shape[1]
    bm = 1024
    grid = (m // bm,)
    return pl.pallas_call(
        _mm_body,
        grid=grid,
        in_specs=[
            pl.BlockSpec((bm, k), lambda i: (i, 0)),
            pl.BlockSpec((k, n), lambda i: (0, 0)),
            pl.BlockSpec((1, n), lambda i: (0, 0)),
        ],
        out_specs=pl.BlockSpec((bm, n), lambda i: (i, 0)),
        out_shape=jax.ShapeDtypeStruct((m, n), jnp.float32),
    )(x, w, b.reshape(1, n))


def _gcn_conv(x, src, dst, ew, dinv, W, b):
    norm = dinv[src] * ew * dinv[dst]
    xw = x @ W
    out = jnp.zeros_like(xw).at[dst].add(xw[src] * norm[:, None])
    out = out + xw * (dinv * dinv)[:, None]
    return out + b


def _gatv2_conv(x, edge_index, Wl, Wr, att, b, heads, out_ch, num_nodes):
    src = edge_index[0]
    dst = edge_index[1]
    loop = jnp.arange(num_nodes, dtype=src.dtype)
    src = jnp.concatenate([src, loop])
    dst = jnp.concatenate([dst, loop])
    xl = (x @ Wl).reshape(num_nodes, heads, out_ch)
    xr = (x @ Wr).reshape(num_nodes, heads, out_ch)
    e = jax.nn.leaky_relu(xl[src] + xr[dst], negative_slope=0.2)
    alpha = (e * att[None, :, :]).sum(-1)
    amax = jnp.full((num_nodes, heads), -jnp.inf, dtype=alpha.dtype).at[dst].max(alpha)
    ex = jnp.exp(alpha - amax[dst])
    denom = jnp.zeros((num_nodes, heads), dtype=alpha.dtype).at[dst].add(ex)
    a = ex / (denom[dst] + 1e-16)
    out = jnp.zeros((num_nodes, heads, out_ch), dtype=x.dtype).at[dst].add(xl[src] * a[:, :, None])
    return out.mean(axis=1) + b


def kernel(x_m, x_d, Data_M_m, Data_M_d, edges_m, edges_d, edge_index,
           W_m1, b_m1, W_m2, b_m2, W_d1, b_d1, W_d2, b_d2,
           Wl1, Wr1, att1, bg1, Wl2, Wr2, att2, bg2, Wjk, bjk):
    srcm = edges_m[0].reshape(E_M // 128, 128)
    dstm = edges_m[1].reshape(E_M // 128, 128)
    srcd = edges_d[0].reshape(E_D // 128, 128)
    dstd = edges_d[1].reshape(E_D // 128, 128)
    ew_m2, ew_d2, degp = _sc_ew_deg(
        Data_M_m.reshape(N_M * N_M), Data_M_d.reshape(N_D * N_D),
        srcm, dstm, srcd, dstd)
    dinv_m = _tc_dinv(degp[0]).reshape(N_M)
    dinv_d = _tc_dinv(degp[1]).reshape(N_D)
    d2m = (dinv_m * dinv_m)[:, None]
    d2d = (dinv_d * dinv_d)[:, None]
    z256 = jnp.zeros((HIDDEN,), jnp.float32)

    xw1m = _tc_matmul_bias(x_m, W_m1, z256)
    xw1d = _tc_matmul_bias(x_d, W_d1, z256)
    p1 = _sc_gcn_agg(xw1m, xw1d, srcm, dstm, ew_m2, srcd, dstd, ew_d2,
                     dinv_m, dinv_d)
    h_m = jax.nn.relu(p1[0] + xw1m * d2m + b_m1)
    h_d = jax.nn.relu(p1[1] + xw1d * d2d + b_d1)
    xw2m = _tc_matmul_bias(h_m, W_m2, z256)
    xw2d = _tc_matmul_bias(h_d, W_d2, z256)
    p2 = _sc_gcn_agg(xw2m, xw2d, srcm, dstm, ew_m2, srcd, dstd, ew_d2,
                     dinv_m, dinv_d)
    mm2 = p2[0] + xw2m * d2m + b_m2
    dd2 = p2[1] + xw2d * d2d + b_d2
    x = jnp.concatenate([jax.nn.relu(mm2), jax.nn.relu(dd2)], axis=0)
    jk0 = x

    loop = jnp.arange(N, dtype=jnp.int32)
    srcg = jnp.concatenate([edge_index[0], loop]).reshape((E + N) // 32, 32)
    dstg = jnp.concatenate([edge_index[1], loop]).reshape((E + N) // 32, 32)

    xl1 = _tc_matmul_bias(x, Wl1, jnp.zeros((HEADS * C1,), jnp.float32))
    xr1 = _tc_matmul_bias(x, Wr1, jnp.zeros((HEADS * C1,), jnp.float32))
    g1 = _sc_gat(xl1, xr1, att1.reshape(HEADS * C1), srcg, dstg, C1)
    jk1 = jax.nn.elu(g1 * (1.0 / HEADS) + bg1)

    xl2 = _tc_matmul_bias(jk1, Wl2, jnp.zeros((HEADS * C2,), jnp.float32))
    xr2 = _tc_matmul_bias(jk1, Wr2, jnp.zeros((HEADS * C2,), jnp.float32))
    g2 = _sc_gat2(xl2, xr2, att2.reshape(HEADS * C2), srcg, dstg)
    x2 = g2.reshape(N, C2) * (1.0 / HEADS) + bg2

    xcat = jnp.concatenate([jk0, jk1, x2], axis=1)
    return _tc_matmul_bias(xcat, Wjk, bjk)


# speedup vs baseline: 1.0419x; 1.0419x over previous
import functools
import jax
import jax.numpy as jnp
from jax import lax
from jax.experimental import pallas as pl
from jax.experimental.pallas import tpu as pltpu
from jax.experimental.pallas import tpu_sc as plsc

N_M = 4096
N_D = 4096
N = N_M + N_D
HIDDEN = 256
HEADS = 4
C1 = 128
C2 = 64
E_M = 131072
E_D = 131072
E = 262144


_SC_MESH = plsc.VectorSubcoreMesh(core_axis_name="c", subcore_axis_name="s")
_SC_PARAMS = pltpu.CompilerParams(needs_layout_passes=False)


def _ew_deg_body(mm_ref, md_ref, srcm_ref, dstm_ref, srcd_ref, dstd_ref,
                 ewm_ref, ewd_ref, degp_ref,
                 src_b, dst_b, idx_b, ew_b, deg_l, sem):
    cid = lax.axis_index("c")
    sid = lax.axis_index("s")
    z16 = jnp.zeros((16,), jnp.float32)

    def zbody(i, _):
        deg_l[pl.ds(i * 16, 16)] = z16
        return 0
    lax.fori_loop(0, N_M // 16, zbody, 0)

    def _process(m_ref, src_ref, dst_ref, ew_ref):
        base_r = sid * 64
        pltpu.sync_copy(src_ref.at[pl.ds(base_r, 64)], src_b)
        pltpu.sync_copy(dst_ref.at[pl.ds(base_r, 64)], dst_b)

        def ibody(j, _):
            for k in range(8):
                s16 = src_b[j, pl.ds(k * 16, 16)]
                d16 = dst_b[j, pl.ds(k * 16, 16)]
                idx_b[j, pl.ds(k * 16, 16)] = s16 * N_M + d16
            return 0
        lax.fori_loop(0, 64, ibody, 0)

        def gbody(j, _):
            pltpu.async_copy(m_ref.at[idx_b.at[j]], ew_b.at[j], sem).wait()
            return 0
        lax.fori_loop(0, 64, gbody, 0)

        pltpu.sync_copy(ew_b, ew_ref.at[pl.ds(base_r, 64)])

        def dbody(j, _):
            for k in range(8):
                d16 = dst_b[j, pl.ds(k * 16, 16)]
                e16 = ew_b[j, pl.ds(k * 16, 16)]
                plsc.addupdate_scatter(deg_l, [d16], e16)
            return 0
        lax.fori_loop(0, 64, dbody, 0)

    plsc.subcore_barrier()

    @pl.when(cid == 0)
    def _():
        _process(mm_ref, srcm_ref, dstm_ref, ewm_ref)

    @pl.when(cid == 1)
    def _():
        _process(md_ref, srcd_ref, dstd_ref, ewd_ref)

    pltpu.sync_copy(deg_l, degp_ref.at[cid, sid])


def _sc_ew_deg(mm_flat, md_flat, srcm, dstm, srcd, dstd):
    f = pl.kernel(
        _ew_deg_body,
        out_type=(
            jax.ShapeDtypeStruct((E_M // 128, 128), jnp.float32),
            jax.ShapeDtypeStruct((E_D // 128, 128), jnp.float32),
            jax.ShapeDtypeStruct((2, 16, N_M), jnp.float32),
        ),
        mesh=_SC_MESH,
        scratch_types=[
            pltpu.VMEM((64, 128), jnp.int32),
            pltpu.VMEM((64, 128), jnp.int32),
            pltpu.VMEM((64, 128), jnp.int32),
            pltpu.VMEM((64, 128), jnp.float32),
            pltpu.VMEM((N_M,), jnp.float32),
            pltpu.SemaphoreType.DMA,
        ],
        compiler_params=_SC_PARAMS,
    )
    return f(mm_flat, md_flat, srcm, dstm, srcd, dstd)


_CAP = 10240


def _gcn_agg_body(xwm_ref, xwd_ref, srcm_ref, dstm_ref, ewm_ref,
                  srcd_ref, dstd_ref, ewd_ref, dinvm_ref, dinvd_ref,
                  part_ref,
                  src_b, dst_b, ew_b, sloc, dloc, nloc, dinv_l, rows, rows2,
                  acc, sem, sem2):
    cid = lax.axis_index("c")
    sid = lax.axis_index("s")
    z16 = jnp.zeros((16,), jnp.float32)
    zi16 = jnp.zeros((16,), jnp.int32)

    def zb(i, _):
        sloc[pl.ds(i * 16, 16)] = zi16
        dloc[pl.ds(i * 16, 16)] = zi16
        nloc[pl.ds(i * 16, 16)] = z16
        return 0
    lax.fori_loop(0, _CAP // 16, zb, 0)

    def za(r, _):
        for k in range(HIDDEN // 16):
            acc[r, pl.ds(k * 16, 16)] = z16
        return 0
    lax.fori_loop(0, 256, za, 0)

    def _process(xw_ref, src_ref, dst_ref, ew_ref, dinv_ref):
        pltpu.sync_copy(dinv_ref, dinv_l)
        lo = sid * 256

        def obody(o, off):
            d1 = pltpu.async_copy(src_ref.at[pl.ds(o * 32, 32)], src_b, sem)
            d2 = pltpu.async_copy(dst_ref.at[pl.ds(o * 32, 32)], dst_b, sem2)
            d3 = pltpu.async_copy(ew_ref.at[pl.ds(o * 32, 32)], ew_b, sem)
            d1.wait()
            d2.wait()
            d3.wait()

            def cbody(g, off):
                j = g // 8
                k = g % 8
                s16 = src_b[j, pl.ds(k * 16, 16)]
                d16 = dst_b[j, pl.ds(k * 16, 16)]
                e16 = ew_b[j, pl.ds(k * 16, 16)]
                msk = (d16 >> 8) == sid
                plsc.store_compressed(sloc.at[pl.ds(off, 16)], s16, mask=msk)
                plsc.store_compressed(dloc.at[pl.ds(off, 16)], d16 & 255, mask=msk)
                plsc.store_compressed(nloc.at[pl.ds(off, 16)], e16, mask=msk)
                cnt = plsc.all_reduce_population_count(msk)[0]
                return off + cnt
            return lax.fori_loop(0, 256, cbody, off)
        ne = lax.fori_loop(0, 32, obody, 0)

        nb = (ne + 15) >> 4

        def nbody(g, _):
            s16 = sloc[pl.ds(g * 16, 16)]
            d16 = dloc[pl.ds(g * 16, 16)] + lo
            e16 = nloc[pl.ds(g * 16, 16)]
            di_s = plsc.load_gather(dinv_l, [s16])
            di_d = plsc.load_gather(dinv_l, [d16])
            nloc[pl.ds(g * 16, 16)] = di_s * e16 * di_d
            return 0
        lax.fori_loop(0, nb, nbody, 0)

        nbat = (ne + 31) >> 5

        def bproc(buf, b):
            def gbody(g, _):
                n16 = nloc[pl.ds(b * 32 + g * 16, 16)]
                d16 = dloc[pl.ds(b * 32 + g * 16, 16)]
                for i in range(16):
                    r = g * 16 + i
                    s = n16[i]
                    dl = d16[i]
                    for k in range(HIDDEN // 16):
                        acc[dl, pl.ds(k * 16, 16)] = (
                            acc[dl, pl.ds(k * 16, 16)]
                            + buf[r, pl.ds(k * 16, 16)] * s)
                return 0
            lax.fori_loop(0, 2, gbody, 0)

        @pl.when(nbat > 0)
        def _():
            pltpu.async_copy(xw_ref.at[sloc.at[pl.ds(0, 32)]], rows, sem)

        def bpair(p, _):
            b0 = 2 * p
            b1 = b0 + 1

            @pl.when(b1 < nbat)
            def _():
                pltpu.async_copy(
                    xw_ref.at[sloc.at[pl.ds(b1 * 32, 32)]], rows2, sem2)
            pltpu.make_async_copy(
                xw_ref.at[sloc.at[pl.ds(b0 * 32, 32)]], rows, sem).wait()
            bproc(rows, b0)

            @pl.when(b1 < nbat)
            def _():
                @pl.when(b1 + 1 < nbat)
                def _():
                    pltpu.async_copy(
                        xw_ref.at[sloc.at[pl.ds((b1 + 1) * 32, 32)]],
                        rows, sem)
                pltpu.make_async_copy(
                    xw_ref.at[sloc.at[pl.ds(b1 * 32, 32)]], rows2, sem2).wait()
                bproc(rows2, b1)
            return 0
        lax.fori_loop(0, (nbat + 1) >> 1, bpair, 0)

    @pl.when(cid == 0)
    def _():
        _process(xwm_ref, srcm_ref, dstm_ref, ewm_ref, dinvm_ref)

    @pl.when(cid == 1)
    def _():
        _process(xwd_ref, srcd_ref, dstd_ref, ewd_ref, dinvd_ref)

    pltpu.sync_copy(acc, part_ref.at[cid, pl.ds(sid * 256, 256)])


def _sc_gcn_agg(xw_m, xw_d, srcm, dstm, ewm, srcd, dstd, ewd, dinv_m, dinv_d):
    f = pl.kernel(
        _gcn_agg_body,
        out_type=jax.ShapeDtypeStruct((2, N_M, HIDDEN), jnp.float32),
        mesh=_SC_MESH,
        scratch_types=[
            pltpu.VMEM((32, 128), jnp.int32),
            pltpu.VMEM((32, 128), jnp.int32),
            pltpu.VMEM((32, 128), jnp.float32),
            pltpu.VMEM((_CAP,), jnp.int32),
            pltpu.VMEM((_CAP,), jnp.int32),
            pltpu.VMEM((_CAP,), jnp.float32),
            pltpu.VMEM((N_M,), jnp.float32),
            pltpu.VMEM((32, HIDDEN), jnp.float32),
            pltpu.VMEM((32, HIDDEN), jnp.float32),
            pltpu.VMEM((256, HIDDEN), jnp.float32),
            pltpu.SemaphoreType.DMA,
            pltpu.SemaphoreType.DMA,
        ],
        compiler_params=_SC_PARAMS,
    )
    return f(xw_m, xw_d, srcm, dstm, ewm, srcd, dstd, ewd, dinv_m, dinv_d)


_GCAP = 9216


def _make_gat_body(C):
    HC = HEADS * C

    def body(xl_ref, xr_ref, att_ref, src_ref, dst_ref, out_ref,
             src_b, dst_b, sloc, dloc, exl, den, att_l, xlr, xrr, acc,
             sem, sem2):
        cid = lax.axis_index("c")
        sid = lax.axis_index("s")
        wid = cid * 16 + sid
        lo = wid * 256
        z16 = jnp.zeros((16,), jnp.float32)
        zi16 = jnp.zeros((16,), jnp.int32)
        ninf = jnp.full((16,), -1e30, jnp.float32)
        lane0 = lax.iota(jnp.int32, 16) == 0

        pltpu.sync_copy(att_ref, att_l)

        def zb(i, _):
            sloc[pl.ds(i * 16, 16)] = zi16
            dloc[pl.ds(i * 16, 16)] = zi16
            return 0
        lax.fori_loop(0, _GCAP // 16, zb, 0)

        def ze(i, _):
            exl[pl.ds(i * 16, 16)] = ninf
            return 0
        lax.fori_loop(0, 4 * _GCAP // 16, ze, 0)

        def zd(i, _):
            den[pl.ds(i * 16, 16)] = z16
            return 0
        lax.fori_loop(0, 64, zd, 0)

        def za(r, _):
            for k in range(C // 16):
                acc[r, pl.ds(k * 16, 16)] = z16
            return 0
        lax.fori_loop(0, 256, za, 0)

        def obody(o, off):
            d1 = pltpu.async_copy(src_ref.at[pl.ds(o * 64, 64)], src_b, sem)
            d2 = pltpu.async_copy(dst_ref.at[pl.ds(o * 64, 64)], dst_b, sem2)
            d1.wait()
            d2.wait()

            def cbody(g, off):
                j = g // 2
                k = g % 2
                s16 = src_b[j, pl.ds(k * 16, 16)]
                d16 = dst_b[j, pl.ds(k * 16, 16)]
                msk = (d16 >> 8) == wid
                plsc.store_compressed(sloc.at[pl.ds(off, 16)], s16, mask=msk)
                plsc.store_compressed(dloc.at[pl.ds(off, 16)], d16, mask=msk)
                cnt = plsc.all_reduce_population_count(msk)[0]
                return off + cnt
            return lax.fori_loop(0, 128, cbody, off)
        ne = lax.fori_loop(0, 132, obody, 0)

        nbat = (ne + 15) >> 4

        def s1iss(b, bl, br, sm):
            pltpu.async_copy(xl_ref.at[sloc.at[pl.ds(b * 16, 16)]], bl, sm)
            pltpu.async_copy(xr_ref.at[dloc.at[pl.ds(b * 16, 16)]], br, sm)

        def s1drain(b, bl, br, sm):
            pltpu.make_async_copy(
                xl_ref.at[sloc.at[pl.ds(b * 16, 16)]], bl, sm).wait()
            pltpu.make_async_copy(
                xr_ref.at[dloc.at[pl.ds(b * 16, 16)]], br, sm).wait()

        def s1proc(bl, br, b):
            def ebody(r, _):
                e = b * 16 + r
                for h in range(HEADS):
                    t = z16
                    for k in range(C // 16):
                        o = h * C + k * 16
                        z = bl[r, pl.ds(o, 16)] + br[r, pl.ds(o, 16)]
                        l = jnp.maximum(z, z * 0.2)
                        t = t + l * att_l[pl.ds(o, 16)]
                    a = jnp.sum(t)
                    plsc.store_scatter(
                        exl, [jnp.broadcast_to(h * _GCAP + e, (16,))],
                        jnp.broadcast_to(a, (16,)), mask=lane0)
                return 0
            lax.fori_loop(0, 16, ebody, 0)

        def s1body(b, _):
            d1 = pltpu.async_copy(
                xl_ref.at[sloc.at[pl.ds(b * 16, 16)]], xlr, sem)
            d2 = pltpu.async_copy(
                xr_ref.at[dloc.at[pl.ds(b * 16, 16)]], xrr, sem2)
            d1.wait()
            d2.wait()
            s1proc(xlr, xrr, b)
            return 0
        lax.fori_loop(0, nbat, s1body, 0)

        ng = (ne + 15) >> 4

        def exbody(g, _):
            dl16 = dloc[pl.ds(g * 16, 16)] & 255
            for h in range(HEADS):
                ex16 = jnp.exp(exl[pl.ds(h * _GCAP + g * 16, 16)])
                exl[pl.ds(h * _GCAP + g * 16, 16)] = ex16
                plsc.addupdate_scatter(den, [h * 256 + dl16], ex16)
            return 0
        lax.fori_loop(0, ng, exbody, 0)

        def rdbody(g, _):
            d16 = den[pl.ds(g * 16, 16)]
            den[pl.ds(g * 16, 16)] = 1.0 / (d16 + 1e-16)
            return 0
        lax.fori_loop(0, 64, rdbody, 0)

        def s2proc(buf, b):
            base = b * 16
            dl16 = dloc[pl.ds(base, 16)] & 255
            av = []
            for h in range(HEADS):
                rd = plsc.load_gather(den, [h * 256 + dl16])
                av.append(exl[pl.ds(h * _GCAP + base, 16)] * rd)
            for i in range(16):
                dl = dl16[i]
                s0 = av[0][i]
                s1 = av[1][i]
                s2 = av[2][i]
                s3 = av[3][i]
                for k in range(C // 16):
                    acc[dl, pl.ds(k * 16, 16)] = (
                        acc[dl, pl.ds(k * 16, 16)]
                        + buf[i, pl.ds(0 * C + k * 16, 16)] * s0
                        + buf[i, pl.ds(1 * C + k * 16, 16)] * s1
                        + buf[i, pl.ds(2 * C + k * 16, 16)] * s2
                        + buf[i, pl.ds(3 * C + k * 16, 16)] * s3)

        @pl.when(nbat > 0)
        def _():
            pltpu.async_copy(xl_ref.at[sloc.at[pl.ds(0, 16)]], xlr, sem)

        def s2pair(p, _):
            b0 = 2 * p
            b1 = b0 + 1

            @pl.when(b1 < nbat)
            def _():
                pltpu.async_copy(
                    xl_ref.at[sloc.at[pl.ds(b1 * 16, 16)]], xrr, sem2)
            pltpu.make_async_copy(
                xl_ref.at[sloc.at[pl.ds(b0 * 16, 16)]], xlr, sem).wait()
            s2proc(xlr, b0)

            @pl.when(b1 < nbat)
            def _():
                @pl.when(b1 + 1 < nbat)
                def _():
                    pltpu.async_copy(
                        xl_ref.at[sloc.at[pl.ds((b1 + 1) * 16, 16)]],
                        xlr, sem)
                pltpu.make_async_copy(
                    xl_ref.at[sloc.at[pl.ds(b1 * 16, 16)]], xrr, sem2).wait()
                s2proc(xrr, b1)
            return 0
        lax.fori_loop(0, (nbat + 1) >> 1, s2pair, 0)

        pltpu.sync_copy(acc, out_ref.at[pl.ds(lo, 256)])

    return body


def _sc_gat(xl, xr, att, src2, dst2, C):
    f = pl.kernel(
        _make_gat_body(C),
        out_type=jax.ShapeDtypeStruct((N, C), jnp.float32),
        mesh=_SC_MESH,
        scratch_types=[
            pltpu.VMEM((64, 32), jnp.int32),
            pltpu.VMEM((64, 32), jnp.int32),
            pltpu.VMEM((_GCAP,), jnp.int32),
            pltpu.VMEM((_GCAP,), jnp.int32),
            pltpu.VMEM((4 * _GCAP,), jnp.float32),
            pltpu.VMEM((1024,), jnp.float32),
            pltpu.VMEM((HEADS * C,), jnp.float32),
            pltpu.VMEM((16, HEADS * C), jnp.float32),
            pltpu.VMEM((16, HEADS * C), jnp.float32),
            pltpu.VMEM((256, C), jnp.float32),
            pltpu.SemaphoreType.DMA,
            pltpu.SemaphoreType.DMA,
        ],
        compiler_params=_SC_PARAMS,
    )
    return f(xl, xr, att, src2, dst2)


def _rsqrt_body(d_ref, o_ref):
    d = jnp.sum(d_ref[...], axis=0, keepdims=True) + 1.0
    o_ref[...] = jnp.where(d > 0, lax.rsqrt(d), 0.0)


def _tc_dinv(degp):
    return pl.pallas_call(
        _rsqrt_body,
        out_shape=jax.ShapeDtypeStruct((1, N_M), jnp.float32),
    )(degp)


def _mm_body(x_ref, w_ref, b_ref, o_ref):
    o_ref[...] = (
        jnp.dot(x_ref[...], w_ref[...], preferred_element_type=jnp.float32)
        + b_ref[...]
    )


def _tc_matmul_bias(x, w, b):
    m, k = x.shape
    n = w.shape[1]
    bm = 1024
    grid = (m // bm,)
    return pl.pallas_call(
        _mm_body,
        grid=grid,
        in_specs=[
            pl.BlockSpec((bm, k), lambda i: (i, 0)),
            pl.BlockSpec((k, n), lambda i: (0, 0)),
            pl.BlockSpec((1, n), lambda i: (0, 0)),
        ],
        out_specs=pl.BlockSpec((bm, n), lambda i: (i, 0)),
        out_shape=jax.ShapeDtypeStruct((m, n), jnp.float32),
    )(x, w, b.reshape(1, n))


def _gcn_conv(x, src, dst, ew, dinv, W, b):
    norm = dinv[src] * ew * dinv[dst]
    xw = x @ W
    out = jnp.zeros_like(xw).at[dst].add(xw[src] * norm[:, None])
    out = out + xw * (dinv * dinv)[:, None]
    return out + b


def _gatv2_conv(x, edge_index, Wl, Wr, att, b, heads, out_ch, num_nodes):
    src = edge_index[0]
    dst = edge_index[1]
    loop = jnp.arange(num_nodes, dtype=src.dtype)
    src = jnp.concatenate([src, loop])
    dst = jnp.concatenate([dst, loop])
    xl = (x @ Wl).reshape(num_nodes, heads, out_ch)
    xr = (x @ Wr).reshape(num_nodes, heads, out_ch)
    e = jax.nn.leaky_relu(xl[src] + xr[dst], negative_slope=0.2)
    alpha = (e * att[None, :, :]).sum(-1)
    amax = jnp.full((num_nodes, heads), -jnp.inf, dtype=alpha.dtype).at[dst].max(alpha)
    ex = jnp.exp(alpha - amax[dst])
    denom = jnp.zeros((num_nodes, heads), dtype=alpha.dtype).at[dst].add(ex)
    a = ex / (denom[dst] + 1e-16)
    out = jnp.zeros((num_nodes, heads, out_ch), dtype=x.dtype).at[dst].add(xl[src] * a[:, :, None])
    return out.mean(axis=1) + b


def kernel(x_m, x_d, Data_M_m, Data_M_d, edges_m, edges_d, edge_index,
           W_m1, b_m1, W_m2, b_m2, W_d1, b_d1, W_d2, b_d2,
           Wl1, Wr1, att1, bg1, Wl2, Wr2, att2, bg2, Wjk, bjk):
    srcm = edges_m[0].reshape(E_M // 128, 128)
    dstm = edges_m[1].reshape(E_M // 128, 128)
    srcd = edges_d[0].reshape(E_D // 128, 128)
    dstd = edges_d[1].reshape(E_D // 128, 128)
    ew_m2, ew_d2, degp = _sc_ew_deg(
        Data_M_m.reshape(N_M * N_M), Data_M_d.reshape(N_D * N_D),
        srcm, dstm, srcd, dstd)
    dinv_m = _tc_dinv(degp[0]).reshape(N_M)
    dinv_d = _tc_dinv(degp[1]).reshape(N_D)
    d2m = (dinv_m * dinv_m)[:, None]
    d2d = (dinv_d * dinv_d)[:, None]
    z256 = jnp.zeros((HIDDEN,), jnp.float32)

    xw1m = _tc_matmul_bias(x_m, W_m1, z256)
    xw1d = _tc_matmul_bias(x_d, W_d1, z256)
    p1 = _sc_gcn_agg(xw1m, xw1d, srcm, dstm, ew_m2, srcd, dstd, ew_d2,
                     dinv_m, dinv_d)
    h_m = jax.nn.relu(p1[0] + xw1m * d2m + b_m1)
    h_d = jax.nn.relu(p1[1] + xw1d * d2d + b_d1)
    xw2m = _tc_matmul_bias(h_m, W_m2, z256)
    xw2d = _tc_matmul_bias(h_d, W_d2, z256)
    p2 = _sc_gcn_agg(xw2m, xw2d, srcm, dstm, ew_m2, srcd, dstd, ew_d2,
                     dinv_m, dinv_d)
    mm2 = p2[0] + xw2m * d2m + b_m2
    dd2 = p2[1] + xw2d * d2d + b_d2
    x = jnp.concatenate([jax.nn.relu(mm2), jax.nn.relu(dd2)], axis=0)
    jk0 = x

    loop = jnp.arange(N, dtype=jnp.int32)
    srcg = jnp.concatenate([edge_index[0], loop]).reshape((E + N) // 32, 32)
    dstg = jnp.concatenate([edge_index[1], loop]).reshape((E + N) // 32, 32)

    xl1 = _tc_matmul_bias(x, Wl1, jnp.zeros((HEADS * C1,), jnp.float32))
    xr1 = _tc_matmul_bias(x, Wr1, jnp.zeros((HEADS * C1,), jnp.float32))
    g1 = _sc_gat(xl1, xr1, att1.reshape(HEADS * C1), srcg, dstg, C1)
    jk1 = jax.nn.elu(g1 * (1.0 / HEADS) + bg1)

    xl2 = _tc_matmul_bias(jk1, Wl2, jnp.zeros((HEADS * C2,), jnp.float32))
    xr2 = _tc_matmul_bias(jk1, Wr2, jnp.zeros((HEADS * C2,), jnp.float32))
    g2 = _sc_gat(xl2, xr2, att2.reshape(HEADS * C2), srcg, dstg, C2)
    x2 = g2 * (1.0 / HEADS) + bg2

    xcat = jnp.concatenate([jk0, jk1, x2], axis=1)
    return _tc_matmul_bias(xcat, Wjk, bjk)
